# Initial kernel scaffold; baseline (speedup 1.0000x reference)
import jax
import jax.numpy as jnp
from jax.experimental import pallas as pl


def kernel(src, emb_table, W):
    raise NotImplementedError("write your pallas kernel here")


# speedup vs baseline: 10.5334x; 10.5334x over previous
import functools

import jax
import jax.numpy as jnp
from jax import lax
from jax.experimental import pallas as pl
from jax.experimental.pallas import tpu as pltpu
from jax.experimental.pallas import tpu_sc as plsc

_NUM = 1000000
_RANK = 16
_OUT_DIM = 64
_B = 16384
_L = 50
_N = _B * _L

_NC = 2
_NS = 16
_NW = _NC * _NS
_ROWS_PER_W = _N // _NW
_CHUNK = 3200
_NCHUNKS = _ROWS_PER_W // _CHUNK


def _sc_gather(emb_table, idx):
    mesh = plsc.VectorSubcoreMesh(
        core_axis_name="c", subcore_axis_name="s", num_cores=_NC,
        num_subcores=_NS)

    @functools.partial(
        pl.kernel,
        out_type=jax.ShapeDtypeStruct((_N, _RANK), jnp.float32),
        mesh=mesh,
        scratch_types=[
            pltpu.VMEM((_CHUNK,), jnp.int32),
            pltpu.VMEM((_CHUNK, _RANK), jnp.float32),
            pltpu.SemaphoreType.DMA,
        ],
        compiler_params=pltpu.CompilerParams(use_tc_tiling_on_sc=False),
    )
    def gather_kernel(table_hbm, idx_hbm, out_hbm, idx_v, rows_v, sem):
        wid = lax.axis_index("s") * _NC + lax.axis_index("c")
        base = wid * _ROWS_PER_W
        for c in range(_NCHUNKS):
            off = base + c * _CHUNK
            pltpu.sync_copy(idx_hbm.at[pl.ds(off, _CHUNK)], idx_v)
            pltpu.async_copy(table_hbm.at[idx_v], rows_v, sem).wait()
            pltpu.sync_copy(rows_v, out_hbm.at[pl.ds(off, _CHUNK)])

    return gather_kernel(emb_table, idx)


_TC_BLOCK = 4096


def _tc_matmul_kernel(emb_ref, w_ref, out_ref):
    out_ref[...] = lax.dot_general(
        emb_ref[...], w_ref[...],
        dimension_numbers=(((1,), (1,)), ((), ())),
        preferred_element_type=jnp.float32)


def _tc_project(gathered, W):
    grid = _N // _TC_BLOCK
    return pl.pallas_call(
        _tc_matmul_kernel,
        grid=(grid,),
        in_specs=[
            pl.BlockSpec((_TC_BLOCK, _RANK), lambda i: (i, 0)),
            pl.BlockSpec((_OUT_DIM, _RANK), lambda i: (0, 0)),
        ],
        out_specs=pl.BlockSpec((_TC_BLOCK, _OUT_DIM), lambda i: (i, 0)),
        out_shape=jax.ShapeDtypeStruct((_N, _OUT_DIM), jnp.float32),
    )(gathered, W)


def kernel(src, emb_table, W):
    idx = src.reshape(-1)
    gathered = _sc_gather(emb_table, idx)
    out = _tc_project(gathered, W)
    return out.reshape(_B, _L, _OUT_DIM)


# speedup vs baseline: 12.3916x; 1.1764x over previous
import functools

import jax
import jax.numpy as jnp
from jax import lax
from jax.experimental import pallas as pl
from jax.experimental.pallas import tpu as pltpu
from jax.experimental.pallas import tpu_sc as plsc

_NUM = 1000000
_RANK = 16
_OUT_DIM = 64
_B = 16384
_L = 50
_N = _B * _L
_PACK = 128 // _RANK

_NC = 2
_NS = 16
_NW = _NC * _NS
_ROWS_PER_W = _N // _NW
_CHUNK = 3200
_NCHUNKS = _ROWS_PER_W // _CHUNK
_PACKED_ROWS = _N // _PACK
_CHUNK_PACKED = _CHUNK // _PACK


def _sc_gather(emb_table, idx):
    mesh = plsc.VectorSubcoreMesh(
        core_axis_name="c", subcore_axis_name="s", num_cores=_NC,
        num_subcores=_NS)

    @functools.partial(
        pl.kernel,
        out_type=jax.ShapeDtypeStruct((_N, _RANK), jnp.float32),
        mesh=mesh,
        scratch_types=[
            pltpu.VMEM((_CHUNK,), jnp.int32),
            pltpu.VMEM((_CHUNK, _RANK), jnp.float32),
            pltpu.SemaphoreType.DMA,
        ],
        compiler_params=pltpu.CompilerParams(use_tc_tiling_on_sc=False),
    )
    def gather_kernel(table_hbm, idx_hbm, out_hbm, idx_v, rows_v, sem):
        wid = lax.axis_index("s") * _NC + lax.axis_index("c")
        base = wid * _ROWS_PER_W
        for c in range(_NCHUNKS):
            off = base + c * _CHUNK
            pltpu.sync_copy(idx_hbm.at[pl.ds(off, _CHUNK)], idx_v)
            pltpu.async_copy(table_hbm.at[idx_v], rows_v, sem).wait()
            pltpu.sync_copy(rows_v, out_hbm.at[pl.ds(off, _CHUNK)])

    return gather_kernel(emb_table, idx)


_TC_BLOCK_B = 64


def _tc_matmul_kernel(emb_ref, w_ref, out_ref):
    emb = emb_ref[...]
    rs = [
        lax.dot_general(
            emb, w_ref[j],
            dimension_numbers=(((1,), (0,)), ((), ())),
            preferred_element_type=jnp.float32)
        for j in range(_PACK)
    ]
    r = jnp.stack(rs, axis=1)
    r = r.reshape(_TC_BLOCK_B * _L, _OUT_DIM)
    out_ref[...] = r.reshape(_TC_BLOCK_B, _L, _OUT_DIM)


def _tc_project(gathered, W):
    eye = jnp.eye(_PACK, dtype=W.dtype)
    w8 = jnp.einsum('jk,ro->jkro', eye, W.T).reshape(_PACK, 128, _OUT_DIM)
    grid = _B // _TC_BLOCK_B
    return pl.pallas_call(
        _tc_matmul_kernel,
        grid=(grid,),
        in_specs=[
            pl.BlockSpec((_CHUNK_PACKED, 128), lambda i: (i, 0)),
            pl.BlockSpec((_PACK, 128, _OUT_DIM), lambda i: (0, 0, 0)),
        ],
        out_specs=pl.BlockSpec(
            (_TC_BLOCK_B, _L, _OUT_DIM), lambda i: (i, 0, 0)),
        out_shape=jax.ShapeDtypeStruct((_B, _L, _OUT_DIM), jnp.float32),
    )(gathered, w8)


def kernel(src, emb_table, W):
    idx = src.reshape(-1)
    gathered = _sc_gather(emb_table, idx)
    packed = gathered.reshape(_PACKED_ROWS, 128)
    return _tc_project(packed, W)


# speedup vs baseline: 18.0162x; 1.4539x over previous
import functools

import jax
import jax.numpy as jnp
from jax import lax
from jax.experimental import pallas as pl
from jax.experimental.pallas import tpu as pltpu
from jax.experimental.pallas import tpu_sc as plsc

_NUM = 1000000
_RANK = 16
_OUT_DIM = 64
_B = 16384
_L = 50
_N = _B * _L
_PACK = 128 // _RANK
_NP = _N // _PACK

_NC = 2
_NS = 16
_NW = _NC * _NS
_PROWS_PER_W = _NP // _NW
_PCHUNK = 400
_NCHUNKS = _PROWS_PER_W // _PCHUNK


def _sc_gather(emb_table, idx):
    mesh = plsc.VectorSubcoreMesh(
        core_axis_name="c", subcore_axis_name="s", num_cores=_NC,
        num_subcores=_NS)

    @functools.partial(
        pl.kernel,
        out_type=jax.ShapeDtypeStruct((_NP, 128), jnp.float32),
        mesh=mesh,
        scratch_types=[
            pltpu.VMEM((_PACK, _PCHUNK), jnp.int32),
            pltpu.VMEM((_PACK, _PCHUNK, _RANK), jnp.float32),
            pltpu.SemaphoreType.DMA,
        ],
        compiler_params=pltpu.CompilerParams(use_tc_tiling_on_sc=False),
    )
    def gather_kernel(table_hbm, idx_hbm, out_hbm, idx_v, rows_v, sem):
        wid = lax.axis_index("s") * _NC + lax.axis_index("c")
        base = wid * _PROWS_PER_W
        for c in range(_NCHUNKS):
            p0 = base + c * _PCHUNK
            for j in range(_PACK):
                pltpu.sync_copy(
                    idx_hbm.at[pl.ds(j * _NP + p0, _PCHUNK)], idx_v.at[j])
            for j in range(_PACK):
                pltpu.async_copy(
                    table_hbm.at[idx_v.at[j]], rows_v.at[j], sem).wait()
            for j in range(_PACK):
                pltpu.sync_copy(
                    rows_v.at[j],
                    out_hbm.at[pl.ds(p0, _PCHUNK),
                               pl.ds(j * _RANK, _RANK)])

    return gather_kernel(emb_table, idx)


_TC_ROWS = 1600
_BPS = _TC_ROWS // _L


def _tc_matmul_kernel(emb_ref, w_ref, out_ref):
    emb = emb_ref[...]
    for j in range(_PACK):
        r = lax.dot_general(
            emb, w_ref[j],
            dimension_numbers=(((1,), (0,)), ((), ())),
            preferred_element_type=jnp.float32)
        out_ref[j] = r.reshape(_BPS, _L, _OUT_DIM)


def _tc_project(packed, W):
    eye = jnp.eye(_PACK, dtype=W.dtype)
    w8 = jnp.einsum('jk,ro->jkro', eye, W.T).reshape(_PACK, 128, _OUT_DIM)
    grid_i = _NP // _TC_ROWS
    out4 = pl.pallas_call(
        _tc_matmul_kernel,
        grid=(grid_i,),
        in_specs=[
            pl.BlockSpec((_TC_ROWS, 128), lambda i: (i, 0)),
            pl.BlockSpec((_PACK, 128, _OUT_DIM), lambda i: (0, 0, 0)),
        ],
        out_specs=pl.BlockSpec(
            (_PACK, _BPS, _L, _OUT_DIM), lambda i: (0, i, 0, 0)),
        out_shape=jax.ShapeDtypeStruct(
            (_PACK, _B // _PACK, _L, _OUT_DIM), jnp.float32),
    )(packed, w8)
    return out4.reshape(_B, _L, _OUT_DIM)


def kernel(src, emb_table, W):
    idx = src.reshape(-1)
    packed = _sc_gather(emb_table, idx)
    return _tc_project(packed, W)
